# baseline (speedup 1.0000x reference)
import functools

import jax
import jax.numpy as jnp
from jax import lax
from jax.experimental import pallas as pl
from jax.experimental.pallas import tpu as pltpu
from jax.experimental.pallas import tpu_sc as plsc

E = 8
K = 2
D = 1024
H = 1024
B = 2048

T = 256
MT = 24
P = MT * T
NC = 2
NS = 16
NW = NC * NS
TPW = B // NW
NEG = -1e30


def _gating_body(x_ref, gw_ref, gb_ref, probs_ref, tki_ref, pos_ref,
                 g16_ref, xbf_ref, meta_ref):
    x = x_ref[...]
    logits = jnp.dot(x, gw_ref[...], preferred_element_type=jnp.float32)
    logits = logits + gb_ref[...][None, :]
    m = jnp.max(logits, axis=1, keepdims=True)
    ex = jnp.exp(logits - m)
    probs_ref[...] = ex / jnp.sum(ex, axis=1, keepdims=True)

    idx = lax.broadcasted_iota(jnp.int32, (B, E), 1)
    i1 = jnp.min(jnp.where(logits == m, idx, E), axis=1, keepdims=True)
    oh1 = idx == i1
    masked = jnp.where(oh1, NEG, logits)
    m2 = jnp.max(masked, axis=1, keepdims=True)
    i2 = jnp.min(jnp.where(masked == m2, idx, E), axis=1, keepdims=True)
    oh2 = idx == i2
    tki_ref[...] = jnp.concatenate([i1, i2], axis=1)

    b = jnp.exp(m2 - m)
    g1 = 1.0 / (1.0 + b)
    g2 = b / (1.0 + b)
    ones16 = jnp.ones((1, 128), jnp.float32)
    g16_ref[...] = jnp.concatenate([g1 * ones16, g2 * ones16], axis=0)
    xbf_ref[...] = x.astype(jnp.bfloat16)

    cnt = oh1.astype(jnp.int32) + oh2.astype(jnp.int32)
    c = cnt
    s = 1
    while s < B:
        c = c + jnp.concatenate(
            [jnp.zeros((s, E), jnp.int32), c[:-s]], axis=0)
        s *= 2
    counts = c[B - 1:B, :]
    tiles_e = (counts + (T - 1)) // T
    tin = tiles_e
    s = 1
    while s < E:
        tin = tin + jnp.concatenate(
            [jnp.zeros((1, s), jnp.int32), tin[:, :-s]], axis=1)
        s *= 2
    toff = tin - tiles_e
    num_tiles = tin[:, E - 1:E]

    rank1 = jnp.sum(c * oh1, axis=1, keepdims=True) - 1
    rank2 = jnp.sum(c * oh2, axis=1, keepdims=True) - 1
    base1 = jnp.sum(jnp.broadcast_to(toff, (B, E)) * oh1, axis=1,
                    keepdims=True) * T
    base2 = jnp.sum(jnp.broadcast_to(toff, (B, E)) * oh2, axis=1,
                    keepdims=True) * T
    pos_ref[...] = jnp.concatenate([base1 + rank1, base2 + rank2], axis=1)

    miota = lax.broadcasted_iota(jnp.int32, (1, 128), 1)
    eot = jnp.zeros((1, 128), jnp.int32)
    for e in range(E):
        eot = eot + (miota >= toff[0, e]).astype(jnp.int32)
    eot = eot - 1
    real = (miota < num_tiles).astype(jnp.int32)
    xidx = jnp.minimum(miota, num_tiles - 1)
    meta_ref[...] = jnp.concatenate(
        [eot, real, xidx, jnp.zeros((1, 128), jnp.int32)], axis=0)


def _dispatch_body(x_hbm, pos_hbm, g16_hbm, xs_hbm, gs_hbm,
                   xv, idx0v, idx1v, g16v0, g16v1, sem):
    wid = lax.axis_index("s") * NC + lax.axis_index("c")
    base = wid * TPW
    pltpu.sync_copy(pos_hbm.at[0, wid], idx0v)
    pltpu.sync_copy(pos_hbm.at[1, wid], idx1v)
    pltpu.sync_copy(g16_hbm.at[0, wid], g16v0)
    pltpu.sync_copy(g16_hbm.at[1, wid], g16v1)
    pltpu.sync_copy(x_hbm.at[pl.ds(base, TPW)], xv)
    c0 = pltpu.async_copy(xv, xs_hbm.at[idx0v], sem)
    c1 = pltpu.async_copy(xv, xs_hbm.at[idx1v], sem)
    c2 = pltpu.async_copy(g16v0, gs_hbm.at[idx0v], sem)
    c3 = pltpu.async_copy(g16v1, gs_hbm.at[idx1v], sem)
    c0.wait()
    c1.wait()
    c2.wait()
    c3.wait()


def _ffn_body(meta_ref, xs_ref, w1_ref, b1_ref, w2_ref, b2_ref, gs_ref,
              out_ref):
    m = pl.program_id(0)
    hb = pl.program_id(1)

    @pl.when(meta_ref[1, m] == 1)
    def _():
        xt = xs_ref[...]
        w1 = w1_ref[...].astype(jnp.bfloat16)
        h = jnp.dot(xt, w1, preferred_element_type=jnp.float32)
        h = jnp.maximum(h + b1_ref[0][None, :], 0.0).astype(jnp.bfloat16)
        w2 = w2_ref[...].astype(jnp.bfloat16)
        eo = jnp.dot(h, w2, preferred_element_type=jnp.float32)
        g = gs_ref[...][:, 0:1]

        @pl.when(hb == 0)
        def _():
            out_ref[...] = (eo + b2_ref[0][None, :]) * g

        @pl.when(hb != 0)
        def _():
            out_ref[...] += eo * g


CH = 32


def _combine_body(eo_hbm, pos_hbm, out_hbm, av, bv, idx0v, idx1v, sem0, sem1):
    wid = lax.axis_index("s") * NC + lax.axis_index("c")
    base = wid * TPW
    pltpu.sync_copy(pos_hbm.at[0, wid], idx0v)
    pltpu.sync_copy(pos_hbm.at[1, wid], idx1v)
    for h in range(TPW // CH):
        c0 = pltpu.async_copy(
            eo_hbm.at[idx0v.at[pl.ds(CH * h, CH)]], av, sem0)
        c1 = pltpu.async_copy(
            eo_hbm.at[idx1v.at[pl.ds(CH * h, CH)]], bv, sem1)
        c0.wait()
        c1.wait()
        nsl = D // 16

        @plsc.parallel_loop(0, CH * nsl, unroll=8)
        def _add(j):
            r = j // nsl
            sl = pl.ds(16 * (j % nsl), 16)
            av[r, sl] = av[r, sl] + bv[r, sl]

        pltpu.sync_copy(av, out_hbm.at[pl.ds(base + CH * h, CH)])


@jax.jit
def _moe(x, gate_W, gate_b, W1, b1, W2, b2):
    probs, tki, pos, g16, xbf, meta = pl.pallas_call(
        _gating_body,
        out_shape=(
            jax.ShapeDtypeStruct((B, E), jnp.float32),
            jax.ShapeDtypeStruct((B, K), jnp.int32),
            jax.ShapeDtypeStruct((B, K), jnp.int32),
            jax.ShapeDtypeStruct((K * B, 128), jnp.float32),
            jax.ShapeDtypeStruct((B, D), jnp.bfloat16),
            jax.ShapeDtypeStruct((4, 128), jnp.int32),
        ),
    )(x, gate_W, gate_b)

    pos_kw = pos.T.reshape(K, NW, TPW)
    g16_kw = g16.reshape(K, NW, TPW, 128)
    xp = lax.bitcast_convert_type(xbf.reshape(B, D // 2, 2), jnp.float32)

    mesh = plsc.VectorSubcoreMesh(core_axis_name="c", subcore_axis_name="s")
    xs, gs = pl.kernel(
        _dispatch_body,
        out_type=(
            jax.ShapeDtypeStruct((P, D // 2), jnp.float32),
            jax.ShapeDtypeStruct((P, 128), jnp.float32),
        ),
        mesh=mesh,
        scratch_types=[
            pltpu.VMEM((TPW, D // 2), jnp.float32),
            pltpu.VMEM((TPW,), jnp.int32),
            pltpu.VMEM((TPW,), jnp.int32),
            pltpu.VMEM((TPW, 128), jnp.float32),
            pltpu.VMEM((TPW, 128), jnp.float32),
            pltpu.SemaphoreType.DMA,
        ],
    )(xp, pos_kw, g16_kw)

    HB = 2
    eo = pl.pallas_call(
        _ffn_body,
        grid_spec=pltpu.PrefetchScalarGridSpec(
            num_scalar_prefetch=1,
            grid=(MT, HB),
            in_specs=[
                pl.BlockSpec((T, D), lambda m, hb, meta: (meta[2, m], 0)),
                pl.BlockSpec((None, D, H // HB),
                             lambda m, hb, meta: (meta[0, m], 0, hb)),
                pl.BlockSpec((None, 1, H // HB),
                             lambda m, hb, meta: (meta[0, m], 0, hb)),
                pl.BlockSpec((None, H // HB, D),
                             lambda m, hb, meta: (meta[0, m], hb, 0)),
                pl.BlockSpec((None, 1, D),
                             lambda m, hb, meta: (meta[0, m], 0, 0)),
                pl.BlockSpec((T, 128), lambda m, hb, meta: (meta[2, m], 0)),
            ],
            out_specs=pl.BlockSpec((T, D), lambda m, hb, meta: (meta[2, m], 0)),
        ),
        out_shape=jax.ShapeDtypeStruct((P, D), jnp.float32),
    )(meta, lax.bitcast_convert_type(xs, jnp.bfloat16).reshape(P, D),
      W1, b1[:, None, :], W2, b2[:, None, :], gs)

    out = pl.kernel(
        _combine_body,
        out_type=jax.ShapeDtypeStruct((B, D), jnp.float32),
        mesh=mesh,
        scratch_types=[
            pltpu.VMEM((CH, D), jnp.float32),
            pltpu.VMEM((CH, D), jnp.float32),
            pltpu.VMEM((TPW,), jnp.int32),
            pltpu.VMEM((TPW,), jnp.int32),
            pltpu.SemaphoreType.DMA,
            pltpu.SemaphoreType.DMA,
        ],
    )(eo, pos_kw)

    return out, probs, tki


def kernel(x, gate_W, gate_b, W1, b1, W2, b2):
    return _moe(x, gate_W, gate_b, W1, b1, W2, b2)


# speedup vs baseline: 1.1089x; 1.1089x over previous
import functools

import jax
import jax.numpy as jnp
from jax import lax
from jax.experimental import pallas as pl
from jax.experimental.pallas import tpu as pltpu
from jax.experimental.pallas import tpu_sc as plsc

E = 8
K = 2
D = 1024
H = 1024
B = 2048

T = 256
MT = 24
P = MT * T
NC = 2
NS = 16
NW = NC * NS
TPW = B // NW
NEG = -1e30


def _gating_body(x_ref, gw_ref, gb_ref, probs_ref, tki_ref, pos_ref,
                 g16_ref, xbf_ref, meta_ref):
    x = x_ref[...]
    logits = jnp.dot(x, gw_ref[...], preferred_element_type=jnp.float32)
    logits = logits + gb_ref[...][None, :]
    m = jnp.max(logits, axis=1, keepdims=True)
    ex = jnp.exp(logits - m)
    probs_ref[...] = ex / jnp.sum(ex, axis=1, keepdims=True)

    idx = lax.broadcasted_iota(jnp.int32, (B, E), 1)
    i1 = jnp.min(jnp.where(logits == m, idx, E), axis=1, keepdims=True)
    oh1 = idx == i1
    masked = jnp.where(oh1, NEG, logits)
    m2 = jnp.max(masked, axis=1, keepdims=True)
    i2 = jnp.min(jnp.where(masked == m2, idx, E), axis=1, keepdims=True)
    oh2 = idx == i2
    tki_ref[...] = jnp.concatenate([i1, i2], axis=1)

    b = jnp.exp(m2 - m)
    g1 = 1.0 / (1.0 + b)
    g2 = b / (1.0 + b)
    ones16 = jnp.ones((1, 128), jnp.float32)
    g16_ref[...] = jnp.concatenate([g1 * ones16, g2 * ones16], axis=0)
    xbf_ref[...] = x.astype(jnp.bfloat16)

    cnt = oh1.astype(jnp.int32) + oh2.astype(jnp.int32)
    c = cnt
    s = 1
    while s < B:
        c = c + jnp.concatenate(
            [jnp.zeros((s, E), jnp.int32), c[:-s]], axis=0)
        s *= 2
    counts = c[B - 1:B, :]
    tiles_e = (counts + (T - 1)) // T
    tin = tiles_e
    s = 1
    while s < E:
        tin = tin + jnp.concatenate(
            [jnp.zeros((1, s), jnp.int32), tin[:, :-s]], axis=1)
        s *= 2
    toff = tin - tiles_e
    num_tiles = tin[:, E - 1:E]

    rank1 = jnp.sum(c * oh1, axis=1, keepdims=True) - 1
    rank2 = jnp.sum(c * oh2, axis=1, keepdims=True) - 1
    base1 = jnp.sum(jnp.broadcast_to(toff, (B, E)) * oh1, axis=1,
                    keepdims=True) * T
    base2 = jnp.sum(jnp.broadcast_to(toff, (B, E)) * oh2, axis=1,
                    keepdims=True) * T
    pos_ref[...] = jnp.concatenate([base1 + rank1, base2 + rank2], axis=1)

    miota = lax.broadcasted_iota(jnp.int32, (1, 128), 1)
    eot = jnp.zeros((1, 128), jnp.int32)
    for e in range(E):
        eot = eot + (miota >= toff[0, e]).astype(jnp.int32)
    eot = eot - 1
    real = (miota < num_tiles).astype(jnp.int32)
    xidx = jnp.minimum(miota, num_tiles - 1)
    meta_ref[...] = jnp.concatenate(
        [eot, real, xidx, jnp.zeros((1, 128), jnp.int32)], axis=0)


def _dispatch_body(x_hbm, pos_hbm, g16_hbm, xs_hbm, gs_hbm,
                   xv, idx0v, idx1v, g16v0, g16v1, sem):
    wid = lax.axis_index("s") * NC + lax.axis_index("c")
    base = wid * TPW
    pltpu.sync_copy(pos_hbm.at[0, wid], idx0v)
    pltpu.sync_copy(pos_hbm.at[1, wid], idx1v)
    pltpu.sync_copy(g16_hbm.at[0, wid], g16v0)
    pltpu.sync_copy(g16_hbm.at[1, wid], g16v1)
    pltpu.sync_copy(x_hbm.at[pl.ds(base, TPW)], xv)
    c0 = pltpu.async_copy(xv, xs_hbm.at[idx0v], sem)
    c1 = pltpu.async_copy(xv, xs_hbm.at[idx1v], sem)
    c2 = pltpu.async_copy(g16v0, gs_hbm.at[idx0v], sem)
    c3 = pltpu.async_copy(g16v1, gs_hbm.at[idx1v], sem)
    c0.wait()
    c1.wait()
    c2.wait()
    c3.wait()


def _ffn_body(meta_ref, xs_ref, w1_ref, b1_ref, w2_ref, b2_ref, gs_ref,
              out_ref):
    m = pl.program_id(0)

    @pl.when(meta_ref[1, m] == 1)
    def _():
        xt = xs_ref[...]
        w1 = w1_ref[...].astype(jnp.bfloat16)
        h = jnp.dot(xt, w1, preferred_element_type=jnp.float32)
        h = jnp.maximum(h + b1_ref[0][None, :], 0.0).astype(jnp.bfloat16)
        w2 = w2_ref[...].astype(jnp.bfloat16)
        eo = jnp.dot(h, w2, preferred_element_type=jnp.float32)
        out_ref[...] = (eo + b2_ref[0][None, :]) * gs_ref[...][:, 0:1]


CH = 32


def _combine_body(eo_hbm, pos_hbm, out_hbm, av, bv, idx0v, idx1v, sem0, sem1):
    wid = lax.axis_index("s") * NC + lax.axis_index("c")
    base = wid * TPW
    pltpu.sync_copy(pos_hbm.at[0, wid], idx0v)
    pltpu.sync_copy(pos_hbm.at[1, wid], idx1v)
    for h in range(TPW // CH):
        c0 = pltpu.async_copy(
            eo_hbm.at[idx0v.at[pl.ds(CH * h, CH)]], av, sem0)
        c1 = pltpu.async_copy(
            eo_hbm.at[idx1v.at[pl.ds(CH * h, CH)]], bv, sem1)
        c0.wait()
        c1.wait()
        nsl = D // 16

        @plsc.parallel_loop(0, CH * nsl, unroll=8)
        def _add(j):
            r = j // nsl
            sl = pl.ds(16 * (j % nsl), 16)
            av[r, sl] = av[r, sl] + bv[r, sl]

        pltpu.sync_copy(av, out_hbm.at[pl.ds(base + CH * h, CH)])


@jax.jit
def _moe(x, gate_W, gate_b, W1, b1, W2, b2):
    probs, tki, pos, g16, xbf, meta = pl.pallas_call(
        _gating_body,
        out_shape=(
            jax.ShapeDtypeStruct((B, E), jnp.float32),
            jax.ShapeDtypeStruct((B, K), jnp.int32),
            jax.ShapeDtypeStruct((B, K), jnp.int32),
            jax.ShapeDtypeStruct((K * B, 128), jnp.float32),
            jax.ShapeDtypeStruct((B, D), jnp.bfloat16),
            jax.ShapeDtypeStruct((4, 128), jnp.int32),
        ),
    )(x, gate_W, gate_b)

    pos_kw = pos.T.reshape(K, NW, TPW)
    g16_kw = g16.reshape(K, NW, TPW, 128)
    xp = lax.bitcast_convert_type(xbf.reshape(B, D // 2, 2), jnp.float32)

    mesh = plsc.VectorSubcoreMesh(core_axis_name="c", subcore_axis_name="s")
    xs, gs = pl.kernel(
        _dispatch_body,
        out_type=(
            jax.ShapeDtypeStruct((P, D // 2), jnp.float32),
            jax.ShapeDtypeStruct((P, 128), jnp.float32),
        ),
        mesh=mesh,
        scratch_types=[
            pltpu.VMEM((TPW, D // 2), jnp.float32),
            pltpu.VMEM((TPW,), jnp.int32),
            pltpu.VMEM((TPW,), jnp.int32),
            pltpu.VMEM((TPW, 128), jnp.float32),
            pltpu.VMEM((TPW, 128), jnp.float32),
            pltpu.SemaphoreType.DMA,
        ],
    )(xp, pos_kw, g16_kw)

    eo = pl.pallas_call(
        _ffn_body,
        grid_spec=pltpu.PrefetchScalarGridSpec(
            num_scalar_prefetch=1,
            grid=(MT,),
            in_specs=[
                pl.BlockSpec((T, D), lambda m, meta: (meta[2, m], 0)),
                pl.BlockSpec((None, D, H), lambda m, meta: (meta[0, m], 0, 0)),
                pl.BlockSpec((None, 1, H), lambda m, meta: (meta[0, m], 0, 0)),
                pl.BlockSpec((None, H, D), lambda m, meta: (meta[0, m], 0, 0)),
                pl.BlockSpec((None, 1, D), lambda m, meta: (meta[0, m], 0, 0)),
                pl.BlockSpec((T, 128), lambda m, meta: (meta[2, m], 0)),
            ],
            out_specs=pl.BlockSpec((T, D), lambda m, meta: (meta[2, m], 0)),
        ),
        out_shape=jax.ShapeDtypeStruct((P, D), jnp.float32),
    )(meta, lax.bitcast_convert_type(xs, jnp.bfloat16).reshape(P, D),
      W1, b1[:, None, :], W2, b2[:, None, :], gs)

    out = pl.kernel(
        _combine_body,
        out_type=jax.ShapeDtypeStruct((B, D), jnp.float32),
        mesh=mesh,
        scratch_types=[
            pltpu.VMEM((CH, D), jnp.float32),
            pltpu.VMEM((CH, D), jnp.float32),
            pltpu.VMEM((TPW,), jnp.int32),
            pltpu.VMEM((TPW,), jnp.int32),
            pltpu.SemaphoreType.DMA,
            pltpu.SemaphoreType.DMA,
        ],
    )(eo, pos_kw)

    return out, probs, tki


def kernel(x, gate_W, gate_b, W1, b1, W2, b2):
    return _moe(x, gate_W, gate_b, W1, b1, W2, b2)


# speedup vs baseline: 2.8313x; 2.5533x over previous
import functools

import jax
import jax.numpy as jnp
from jax import lax
from jax.experimental import pallas as pl
from jax.experimental.pallas import tpu as pltpu
from jax.experimental.pallas import tpu_sc as plsc

E = 8
K = 2
D = 1024
H = 1024
B = 2048

T = 256
MT = 24
P = MT * T
NC = 2
NS = 16
NW = NC * NS
TPW = B // NW
NEG = -1e30


def _gating_body(x_ref, gw_ref, gb_ref, probs_ref, tki_ref, pos_ref,
                 g16_ref, meta_ref):
    x = x_ref[...]
    logits = jnp.dot(x, gw_ref[...], preferred_element_type=jnp.float32)
    logits = logits + gb_ref[...][None, :]
    m = jnp.max(logits, axis=1, keepdims=True)
    ex = jnp.exp(logits - m)
    probs_ref[...] = ex / jnp.sum(ex, axis=1, keepdims=True)

    idx = lax.broadcasted_iota(jnp.int32, (B, E), 1)
    i1 = jnp.min(jnp.where(logits == m, idx, E), axis=1, keepdims=True)
    oh1 = idx == i1
    masked = jnp.where(oh1, NEG, logits)
    m2 = jnp.max(masked, axis=1, keepdims=True)
    i2 = jnp.min(jnp.where(masked == m2, idx, E), axis=1, keepdims=True)
    oh2 = idx == i2
    tki_ref[...] = jnp.concatenate([i1, i2], axis=1)

    b = jnp.exp(m2 - m)
    g1 = 1.0 / (1.0 + b)
    g2 = b / (1.0 + b)
    ones16 = jnp.ones((1, 128), jnp.float32)
    g16_ref[...] = jnp.concatenate([g1 * ones16, g2 * ones16], axis=0)

    cnt = oh1.astype(jnp.int32) + oh2.astype(jnp.int32)
    c = cnt
    s = 1
    while s < B:
        c = c + jnp.concatenate(
            [jnp.zeros((s, E), jnp.int32), c[:-s]], axis=0)
        s *= 2
    counts = c[B - 1:B, :]
    tiles_e = (counts + (T - 1)) // T
    tin = tiles_e
    s = 1
    while s < E:
        tin = tin + jnp.concatenate(
            [jnp.zeros((1, s), jnp.int32), tin[:, :-s]], axis=1)
        s *= 2
    toff = tin - tiles_e
    num_tiles = tin[:, E - 1:E]

    rank1 = jnp.sum(c * oh1, axis=1, keepdims=True) - 1
    rank2 = jnp.sum(c * oh2, axis=1, keepdims=True) - 1
    base1 = jnp.sum(jnp.broadcast_to(toff, (B, E)) * oh1, axis=1,
                    keepdims=True) * T
    base2 = jnp.sum(jnp.broadcast_to(toff, (B, E)) * oh2, axis=1,
                    keepdims=True) * T
    pos_ref[...] = jnp.concatenate([base1 + rank1, base2 + rank2], axis=1)

    miota = lax.broadcasted_iota(jnp.int32, (1, 128), 1)
    eot = jnp.zeros((1, 128), jnp.int32)
    for e in range(E):
        eot = eot + (miota >= toff[0, e]).astype(jnp.int32)
    eot = eot - 1
    real = (miota < num_tiles).astype(jnp.int32)
    xidx = jnp.minimum(miota, num_tiles - 1)
    meta_ref[...] = jnp.concatenate(
        [eot, real, xidx, jnp.zeros((1, 128), jnp.int32)], axis=0)


def _dispatch_body(x_hbm, pos_hbm, g16_hbm, xs_hbm, gs_hbm,
                   xv, idx0v, idx1v, g16v0, g16v1, sem):
    wid = lax.axis_index("s") * NC + lax.axis_index("c")
    base = wid * TPW
    pltpu.sync_copy(pos_hbm.at[0, wid], idx0v)
    pltpu.sync_copy(pos_hbm.at[1, wid], idx1v)
    pltpu.sync_copy(g16_hbm.at[0, wid], g16v0)
    pltpu.sync_copy(g16_hbm.at[1, wid], g16v1)
    pltpu.sync_copy(x_hbm.at[pl.ds(base, TPW)], xv)
    c0 = pltpu.async_copy(xv, xs_hbm.at[idx0v], sem)
    c1 = pltpu.async_copy(xv, xs_hbm.at[idx1v], sem)
    c2 = pltpu.async_copy(g16v0, gs_hbm.at[idx0v], sem)
    c3 = pltpu.async_copy(g16v1, gs_hbm.at[idx1v], sem)
    c0.wait()
    c1.wait()
    c2.wait()
    c3.wait()


def _ffn_body(meta_ref, xs_ref, w1_ref, b1_ref, w2_ref, b2_ref, gs_ref,
              out_ref):
    m = pl.program_id(0)

    @pl.when(meta_ref[1, m] == 1)
    def _():
        xt = xs_ref[...].astype(jnp.bfloat16)
        w1 = w1_ref[...].astype(jnp.bfloat16)
        h = jnp.dot(xt, w1, preferred_element_type=jnp.float32)
        h = jnp.maximum(h + b1_ref[0][None, :], 0.0).astype(jnp.bfloat16)
        w2 = w2_ref[...].astype(jnp.bfloat16)
        eo = jnp.dot(h, w2, preferred_element_type=jnp.float32)
        out_ref[...] = (eo + b2_ref[0][None, :]) * gs_ref[...][:, 0:1]


CH = 32


def _combine_body(eo_hbm, pos_hbm, out_hbm, av, bv, idx0v, idx1v, sem0, sem1):
    wid = lax.axis_index("s") * NC + lax.axis_index("c")
    base = wid * TPW
    pltpu.sync_copy(pos_hbm.at[0, wid], idx0v)
    pltpu.sync_copy(pos_hbm.at[1, wid], idx1v)
    for h in range(TPW // CH):
        c0 = pltpu.async_copy(
            eo_hbm.at[idx0v.at[pl.ds(CH * h, CH)]], av, sem0)
        c1 = pltpu.async_copy(
            eo_hbm.at[idx1v.at[pl.ds(CH * h, CH)]], bv, sem1)
        c0.wait()
        c1.wait()
        nsl = D // 16

        @plsc.parallel_loop(0, CH * nsl, unroll=8)
        def _add(j):
            r = j // nsl
            sl = pl.ds(16 * (j % nsl), 16)
            av[r, sl] = av[r, sl] + bv[r, sl]

        pltpu.sync_copy(av, out_hbm.at[pl.ds(base + CH * h, CH)])


@jax.jit
def _moe(x, gate_W, gate_b, W1, b1, W2, b2):
    probs, tki, pos, g16, meta = pl.pallas_call(
        _gating_body,
        out_shape=(
            jax.ShapeDtypeStruct((B, E), jnp.float32),
            jax.ShapeDtypeStruct((B, K), jnp.int32),
            jax.ShapeDtypeStruct((B, K), jnp.int32),
            jax.ShapeDtypeStruct((K * B, 128), jnp.float32),
            jax.ShapeDtypeStruct((4, 128), jnp.int32),
        ),
    )(x, gate_W, gate_b)

    pos_kw = pos.T.reshape(K, NW, TPW)
    g16_kw = g16.reshape(K, NW, TPW, 128)

    mesh = plsc.VectorSubcoreMesh(core_axis_name="c", subcore_axis_name="s")
    xs, gs = pl.kernel(
        _dispatch_body,
        out_type=(
            jax.ShapeDtypeStruct((P, D), jnp.float32),
            jax.ShapeDtypeStruct((P, 128), jnp.float32),
        ),
        mesh=mesh,
        scratch_types=[
            pltpu.VMEM((TPW, D), jnp.float32),
            pltpu.VMEM((TPW,), jnp.int32),
            pltpu.VMEM((TPW,), jnp.int32),
            pltpu.VMEM((TPW, 128), jnp.float32),
            pltpu.VMEM((TPW, 128), jnp.float32),
            pltpu.SemaphoreType.DMA,
        ],
    )(x, pos_kw, g16_kw)

    eo = pl.pallas_call(
        _ffn_body,
        grid_spec=pltpu.PrefetchScalarGridSpec(
            num_scalar_prefetch=1,
            grid=(MT,),
            in_specs=[
                pl.BlockSpec((T, D), lambda m, meta: (meta[2, m], 0)),
                pl.BlockSpec((None, D, H), lambda m, meta: (meta[0, m], 0, 0)),
                pl.BlockSpec((None, 1, H), lambda m, meta: (meta[0, m], 0, 0)),
                pl.BlockSpec((None, H, D), lambda m, meta: (meta[0, m], 0, 0)),
                pl.BlockSpec((None, 1, D), lambda m, meta: (meta[0, m], 0, 0)),
                pl.BlockSpec((T, 128), lambda m, meta: (meta[2, m], 0)),
            ],
            out_specs=pl.BlockSpec((T, D), lambda m, meta: (meta[2, m], 0)),
        ),
        out_shape=jax.ShapeDtypeStruct((P, D), jnp.float32),
    )(meta, xs, W1, b1[:, None, :], W2, b2[:, None, :], gs)

    out = pl.kernel(
        _combine_body,
        out_type=jax.ShapeDtypeStruct((B, D), jnp.float32),
        mesh=mesh,
        scratch_types=[
            pltpu.VMEM((CH, D), jnp.float32),
            pltpu.VMEM((CH, D), jnp.float32),
            pltpu.VMEM((TPW,), jnp.int32),
            pltpu.VMEM((TPW,), jnp.int32),
            pltpu.SemaphoreType.DMA,
            pltpu.SemaphoreType.DMA,
        ],
    )(eo, pos_kw)

    return out, probs, tki


def kernel(x, gate_W, gate_b, W1, b1, W2, b2):
    return _moe(x, gate_W, gate_b, W1, b1, W2, b2)


# speedup vs baseline: 2.9307x; 1.0351x over previous
import functools

import jax
import jax.numpy as jnp
from jax import lax
from jax.experimental import pallas as pl
from jax.experimental.pallas import tpu as pltpu
from jax.experimental.pallas import tpu_sc as plsc

E = 8
K = 2
D = 1024
H = 1024
B = 2048

T = 256
MT = 24
P = MT * T
NC = 2
NS = 16
NW = NC * NS
TPW = B // NW
NEG = -1e30


def _gating_body(x_ref, gw_ref, gb_ref, probs_ref, tki_ref, pos_ref,
                 g16_ref, xp_ref, meta_ref):
    x = x_ref[...]
    xb = x.astype(jnp.bfloat16)
    lo = lax.bitcast_convert_type(xb[:, :D // 2], jnp.uint16)
    hi = lax.bitcast_convert_type(xb[:, D // 2:], jnp.uint16)
    packed = lo.astype(jnp.uint32) | (hi.astype(jnp.uint32) << 16)
    xp_ref[...] = lax.bitcast_convert_type(packed, jnp.int32)
    logits = jnp.dot(x, gw_ref[...], preferred_element_type=jnp.float32)
    logits = logits + gb_ref[...][None, :]
    m = jnp.max(logits, axis=1, keepdims=True)
    ex = jnp.exp(logits - m)
    probs_ref[...] = ex / jnp.sum(ex, axis=1, keepdims=True)

    idx = lax.broadcasted_iota(jnp.int32, (B, E), 1)
    i1 = jnp.min(jnp.where(logits == m, idx, E), axis=1, keepdims=True)
    oh1 = idx == i1
    masked = jnp.where(oh1, NEG, logits)
    m2 = jnp.max(masked, axis=1, keepdims=True)
    i2 = jnp.min(jnp.where(masked == m2, idx, E), axis=1, keepdims=True)
    oh2 = idx == i2
    tki_ref[...] = jnp.concatenate([i1, i2], axis=1)

    b = jnp.exp(m2 - m)
    g1 = 1.0 / (1.0 + b)
    g2 = b / (1.0 + b)
    ones16 = jnp.ones((1, 128), jnp.float32)
    g16_ref[...] = jnp.concatenate([g1 * ones16, g2 * ones16], axis=0)

    cnt = oh1.astype(jnp.int32) + oh2.astype(jnp.int32)
    c = cnt
    s = 1
    while s < B:
        c = c + jnp.concatenate(
            [jnp.zeros((s, E), jnp.int32), c[:-s]], axis=0)
        s *= 2
    counts = c[B - 1:B, :]
    tiles_e = (counts + (T - 1)) // T
    tin = tiles_e
    s = 1
    while s < E:
        tin = tin + jnp.concatenate(
            [jnp.zeros((1, s), jnp.int32), tin[:, :-s]], axis=1)
        s *= 2
    toff = tin - tiles_e
    num_tiles = tin[:, E - 1:E]

    rank1 = jnp.sum(c * oh1, axis=1, keepdims=True) - 1
    rank2 = jnp.sum(c * oh2, axis=1, keepdims=True) - 1
    base1 = jnp.sum(jnp.broadcast_to(toff, (B, E)) * oh1, axis=1,
                    keepdims=True) * T
    base2 = jnp.sum(jnp.broadcast_to(toff, (B, E)) * oh2, axis=1,
                    keepdims=True) * T
    pos_ref[...] = jnp.concatenate([base1 + rank1, base2 + rank2], axis=1)

    miota = lax.broadcasted_iota(jnp.int32, (1, 128), 1)
    eot = jnp.zeros((1, 128), jnp.int32)
    for e in range(E):
        eot = eot + (miota >= toff[0, e]).astype(jnp.int32)
    eot = eot - 1
    real = (miota < num_tiles).astype(jnp.int32)
    xidx = jnp.minimum(miota, num_tiles - 1)
    meta_ref[...] = jnp.concatenate(
        [eot, real, xidx, jnp.zeros((1, 128), jnp.int32)], axis=0)


def _dispatch_body(x_hbm, pos_hbm, g16_hbm, xs_hbm, gs_hbm,
                   xv, idx0v, idx1v, g16v0, g16v1, sem):
    wid = lax.axis_index("s") * NC + lax.axis_index("c")
    base = wid * TPW
    pltpu.sync_copy(pos_hbm.at[0, wid], idx0v)
    pltpu.sync_copy(pos_hbm.at[1, wid], idx1v)
    pltpu.sync_copy(g16_hbm.at[0, wid], g16v0)
    pltpu.sync_copy(g16_hbm.at[1, wid], g16v1)
    pltpu.sync_copy(x_hbm.at[pl.ds(base, TPW)], xv)
    c0 = pltpu.async_copy(xv, xs_hbm.at[idx0v], sem)
    c1 = pltpu.async_copy(xv, xs_hbm.at[idx1v], sem)
    c2 = pltpu.async_copy(g16v0, gs_hbm.at[idx0v], sem)
    c3 = pltpu.async_copy(g16v1, gs_hbm.at[idx1v], sem)
    c0.wait()
    c1.wait()
    c2.wait()
    c3.wait()


def _ffn_body(meta_ref, xs_ref, w1_ref, b1_ref, w2_ref, b2_ref, gs_ref,
              out_ref):
    m = pl.program_id(0)

    @pl.when(meta_ref[1, m] == 1)
    def _():
        v = xs_ref[...]
        xlo = lax.bitcast_convert_type(
            (v & 0xFFFF).astype(jnp.uint16), jnp.bfloat16)
        xhi = lax.bitcast_convert_type(
            lax.shift_right_logical(v, 16).astype(jnp.uint16), jnp.bfloat16)
        w1lo = w1_ref[: D // 2, :].astype(jnp.bfloat16)
        w1hi = w1_ref[D // 2:, :].astype(jnp.bfloat16)
        h = (jnp.dot(xlo, w1lo, preferred_element_type=jnp.float32)
             + jnp.dot(xhi, w1hi, preferred_element_type=jnp.float32))
        h = jnp.maximum(h + b1_ref[0][None, :], 0.0).astype(jnp.bfloat16)
        w2 = w2_ref[...].astype(jnp.bfloat16)
        eo = jnp.dot(h, w2, preferred_element_type=jnp.float32)
        out_ref[...] = (eo + b2_ref[0][None, :]) * gs_ref[...][:, 0:1]


CH = 32


def _combine_body(eo_hbm, pos_hbm, out_hbm, av, bv, idx0v, idx1v, sem0, sem1):
    wid = lax.axis_index("s") * NC + lax.axis_index("c")
    base = wid * TPW
    pltpu.sync_copy(pos_hbm.at[0, wid], idx0v)
    pltpu.sync_copy(pos_hbm.at[1, wid], idx1v)
    for h in range(TPW // CH):
        c0 = pltpu.async_copy(
            eo_hbm.at[idx0v.at[pl.ds(CH * h, CH)]], av, sem0)
        c1 = pltpu.async_copy(
            eo_hbm.at[idx1v.at[pl.ds(CH * h, CH)]], bv, sem1)
        c0.wait()
        c1.wait()
        nsl = D // 16

        @plsc.parallel_loop(0, CH * nsl, unroll=8)
        def _add(j):
            r = j // nsl
            sl = pl.ds(16 * (j % nsl), 16)
            av[r, sl] = av[r, sl] + bv[r, sl]

        pltpu.sync_copy(av, out_hbm.at[pl.ds(base + CH * h, CH)])


@jax.jit
def _moe(x, gate_W, gate_b, W1, b1, W2, b2):
    probs, tki, pos, g16, xp, meta = pl.pallas_call(
        _gating_body,
        out_shape=(
            jax.ShapeDtypeStruct((B, E), jnp.float32),
            jax.ShapeDtypeStruct((B, K), jnp.int32),
            jax.ShapeDtypeStruct((B, K), jnp.int32),
            jax.ShapeDtypeStruct((K * B, 128), jnp.float32),
            jax.ShapeDtypeStruct((B, D // 2), jnp.int32),
            jax.ShapeDtypeStruct((4, 128), jnp.int32),
        ),
    )(x, gate_W, gate_b)

    pos_kw = pos.T.reshape(K, NW, TPW)
    g16_kw = g16.reshape(K, NW, TPW, 128)

    mesh = plsc.VectorSubcoreMesh(core_axis_name="c", subcore_axis_name="s")
    xs, gs = pl.kernel(
        _dispatch_body,
        out_type=(
            jax.ShapeDtypeStruct((P, D // 2), jnp.int32),
            jax.ShapeDtypeStruct((P, 128), jnp.float32),
        ),
        mesh=mesh,
        scratch_types=[
            pltpu.VMEM((TPW, D // 2), jnp.int32),
            pltpu.VMEM((TPW,), jnp.int32),
            pltpu.VMEM((TPW,), jnp.int32),
            pltpu.VMEM((TPW, 128), jnp.float32),
            pltpu.VMEM((TPW, 128), jnp.float32),
            pltpu.SemaphoreType.DMA,
        ],
    )(xp, pos_kw, g16_kw)

    eo = pl.pallas_call(
        _ffn_body,
        grid_spec=pltpu.PrefetchScalarGridSpec(
            num_scalar_prefetch=1,
            grid=(MT,),
            in_specs=[
                pl.BlockSpec((T, D // 2), lambda m, meta: (meta[2, m], 0)),
                pl.BlockSpec((None, D, H), lambda m, meta: (meta[0, m], 0, 0)),
                pl.BlockSpec((None, 1, H), lambda m, meta: (meta[0, m], 0, 0)),
                pl.BlockSpec((None, H, D), lambda m, meta: (meta[0, m], 0, 0)),
                pl.BlockSpec((None, 1, D), lambda m, meta: (meta[0, m], 0, 0)),
                pl.BlockSpec((T, 128), lambda m, meta: (meta[2, m], 0)),
            ],
            out_specs=pl.BlockSpec((T, D), lambda m, meta: (meta[2, m], 0)),
        ),
        out_shape=jax.ShapeDtypeStruct((P, D), jnp.float32),
    )(meta, xs, W1, b1[:, None, :], W2, b2[:, None, :], gs)

    out = pl.kernel(
        _combine_body,
        out_type=jax.ShapeDtypeStruct((B, D), jnp.float32),
        mesh=mesh,
        scratch_types=[
            pltpu.VMEM((CH, D), jnp.float32),
            pltpu.VMEM((CH, D), jnp.float32),
            pltpu.VMEM((TPW,), jnp.int32),
            pltpu.VMEM((TPW,), jnp.int32),
            pltpu.SemaphoreType.DMA,
            pltpu.SemaphoreType.DMA,
        ],
    )(eo, pos_kw)

    return out, probs, tki


def kernel(x, gate_W, gate_b, W1, b1, W2, b2):
    return _moe(x, gate_W, gate_b, W1, b1, W2, b2)


# speedup vs baseline: 3.0202x; 1.0305x over previous
import functools

import jax
import jax.numpy as jnp
from jax import lax
from jax.experimental import pallas as pl
from jax.experimental.pallas import tpu as pltpu
from jax.experimental.pallas import tpu_sc as plsc

E = 8
K = 2
D = 1024
H = 1024
B = 2048

T = 256
MT = 24
P = MT * T
NC = 2
NS = 16
NW = NC * NS
TPW = B // NW
NEG = -1e30


def _gating_body(x_ref, gw_ref, gb_ref, probs_ref, tki_ref, pos_ref,
                 g16_ref, xp_ref, meta_ref):
    x = x_ref[...]
    xb = x.astype(jnp.bfloat16)
    lo = lax.bitcast_convert_type(xb[:, :D // 2], jnp.uint16)
    hi = lax.bitcast_convert_type(xb[:, D // 2:], jnp.uint16)
    packed = lo.astype(jnp.uint32) | (hi.astype(jnp.uint32) << 16)
    xp_ref[...] = lax.bitcast_convert_type(packed, jnp.int32)
    logits = jnp.dot(x, gw_ref[...], preferred_element_type=jnp.float32)
    logits = logits + gb_ref[...][None, :]
    m = jnp.max(logits, axis=1, keepdims=True)
    ex = jnp.exp(logits - m)
    probs_ref[...] = ex / jnp.sum(ex, axis=1, keepdims=True)

    idx = lax.broadcasted_iota(jnp.int32, (B, E), 1)
    i1 = jnp.min(jnp.where(logits == m, idx, E), axis=1, keepdims=True)
    oh1 = idx == i1
    masked = jnp.where(oh1, NEG, logits)
    m2 = jnp.max(masked, axis=1, keepdims=True)
    i2 = jnp.min(jnp.where(masked == m2, idx, E), axis=1, keepdims=True)
    oh2 = idx == i2
    tki_ref[...] = jnp.concatenate([i1, i2], axis=1)

    b = jnp.exp(m2 - m)
    g1 = 1.0 / (1.0 + b)
    g2 = b / (1.0 + b)
    ones16 = jnp.ones((1, 128), jnp.float32)
    g16_ref[...] = jnp.concatenate([g1 * ones16, g2 * ones16], axis=0)

    cnt = oh1.astype(jnp.int32) + oh2.astype(jnp.int32)
    c = cnt
    s = 1
    while s < B:
        c = c + jnp.concatenate(
            [jnp.zeros((s, E), jnp.int32), c[:-s]], axis=0)
        s *= 2
    counts = c[B - 1:B, :]
    tiles_e = (counts + (T - 1)) // T
    tin = tiles_e
    s = 1
    while s < E:
        tin = tin + jnp.concatenate(
            [jnp.zeros((1, s), jnp.int32), tin[:, :-s]], axis=1)
        s *= 2
    toff = tin - tiles_e
    num_tiles = tin[:, E - 1:E]

    rank1 = jnp.sum(c * oh1, axis=1, keepdims=True) - 1
    rank2 = jnp.sum(c * oh2, axis=1, keepdims=True) - 1
    base1 = jnp.sum(jnp.broadcast_to(toff, (B, E)) * oh1, axis=1,
                    keepdims=True) * T
    base2 = jnp.sum(jnp.broadcast_to(toff, (B, E)) * oh2, axis=1,
                    keepdims=True) * T
    pos_ref[...] = jnp.concatenate([base1 + rank1, base2 + rank2], axis=1)

    miota = lax.broadcasted_iota(jnp.int32, (1, 128), 1)
    eot = jnp.zeros((1, 128), jnp.int32)
    for e in range(E):
        eot = eot + (miota >= toff[0, e]).astype(jnp.int32)
    eot = eot - 1
    real = (miota < num_tiles).astype(jnp.int32)
    xidx = jnp.minimum(miota, num_tiles - 1)
    meta_ref[...] = jnp.concatenate(
        [eot, real, xidx, jnp.zeros((1, 128), jnp.int32)], axis=0)


def _dispatch_body(x_hbm, pos_hbm, g16_hbm, xs_hbm, gs_hbm,
                   xv, idx0v, idx1v, g16v0, g16v1, sem):
    wid = lax.axis_index("s") * NC + lax.axis_index("c")
    base = wid * TPW
    pltpu.sync_copy(pos_hbm.at[0, wid], idx0v)
    pltpu.sync_copy(pos_hbm.at[1, wid], idx1v)
    pltpu.sync_copy(g16_hbm.at[0, wid], g16v0)
    pltpu.sync_copy(g16_hbm.at[1, wid], g16v1)
    pltpu.sync_copy(x_hbm.at[pl.ds(base, TPW)], xv)
    c0 = pltpu.async_copy(xv, xs_hbm.at[idx0v], sem)
    c1 = pltpu.async_copy(xv, xs_hbm.at[idx1v], sem)
    c2 = pltpu.async_copy(g16v0, gs_hbm.at[idx0v], sem)
    c3 = pltpu.async_copy(g16v1, gs_hbm.at[idx1v], sem)
    c0.wait()
    c1.wait()
    c2.wait()
    c3.wait()


def _ffn_body(meta_ref, xs_ref, w1_ref, b1_ref, w2_ref, b2_ref, gs_ref,
              out_ref):
    m = pl.program_id(0)

    @pl.when(meta_ref[1, m] == 1)
    def _():
        v = xs_ref[...]
        xlo = lax.bitcast_convert_type(
            (v & 0xFFFF).astype(jnp.uint16), jnp.bfloat16)
        xhi = lax.bitcast_convert_type(
            lax.shift_right_logical(v, 16).astype(jnp.uint16), jnp.bfloat16)
        w1lo = w1_ref[: D // 2, :].astype(jnp.bfloat16)
        w1hi = w1_ref[D // 2:, :].astype(jnp.bfloat16)
        h = (jnp.dot(xlo, w1lo, preferred_element_type=jnp.float32)
             + jnp.dot(xhi, w1hi, preferred_element_type=jnp.float32))
        h = jnp.maximum(h + b1_ref[0][None, :], 0.0).astype(jnp.bfloat16)
        w2 = w2_ref[...].astype(jnp.bfloat16)
        eo = jnp.dot(h, w2, preferred_element_type=jnp.float32)
        out_ref[...] = (eo + b2_ref[0][None, :]) * gs_ref[...][:, 0:1]


CH = 16
NCH = TPW // CH


def _combine_body(eo_hbm, pos_hbm, out_hbm, av0, bv0, av1, bv1,
                  idx0v, idx1v, sem0, sem1):
    wid = lax.axis_index("s") * NC + lax.axis_index("c")
    base = wid * TPW
    pltpu.sync_copy(pos_hbm.at[0, wid], idx0v)
    pltpu.sync_copy(pos_hbm.at[1, wid], idx1v)
    avs = [av0, av1]
    bvs = [bv0, bv1]
    nsl = D // 16

    def fire(h):
        c0 = pltpu.async_copy(
            eo_hbm.at[idx0v.at[pl.ds(CH * h, CH)]], avs[h % 2], sem0)
        c1 = pltpu.async_copy(
            eo_hbm.at[idx1v.at[pl.ds(CH * h, CH)]], bvs[h % 2], sem1)
        return c0, c1

    pend = fire(0)
    for h in range(NCH):
        pend[0].wait()
        pend[1].wait()
        if h + 1 < NCH:
            pend = fire(h + 1)
        av = avs[h % 2]
        bv = bvs[h % 2]

        @plsc.parallel_loop(0, CH * nsl, unroll=8)
        def _add(j):
            r = j // nsl
            sl = pl.ds(16 * (j % nsl), 16)
            av[r, sl] = av[r, sl] + bv[r, sl]

        pltpu.sync_copy(av, out_hbm.at[pl.ds(base + CH * h, CH)])


@jax.jit
def _moe(x, gate_W, gate_b, W1, b1, W2, b2):
    probs, tki, pos, g16, xp, meta = pl.pallas_call(
        _gating_body,
        out_shape=(
            jax.ShapeDtypeStruct((B, E), jnp.float32),
            jax.ShapeDtypeStruct((B, K), jnp.int32),
            jax.ShapeDtypeStruct((B, K), jnp.int32),
            jax.ShapeDtypeStruct((K * B, 128), jnp.float32),
            jax.ShapeDtypeStruct((B, D // 2), jnp.int32),
            jax.ShapeDtypeStruct((4, 128), jnp.int32),
        ),
    )(x, gate_W, gate_b)

    pos_kw = pos.T.reshape(K, NW, TPW)
    g16_kw = g16.reshape(K, NW, TPW, 128)

    mesh = plsc.VectorSubcoreMesh(core_axis_name="c", subcore_axis_name="s")
    xs, gs = pl.kernel(
        _dispatch_body,
        out_type=(
            jax.ShapeDtypeStruct((P, D // 2), jnp.int32),
            jax.ShapeDtypeStruct((P, 128), jnp.float32),
        ),
        mesh=mesh,
        scratch_types=[
            pltpu.VMEM((TPW, D // 2), jnp.int32),
            pltpu.VMEM((TPW,), jnp.int32),
            pltpu.VMEM((TPW,), jnp.int32),
            pltpu.VMEM((TPW, 128), jnp.float32),
            pltpu.VMEM((TPW, 128), jnp.float32),
            pltpu.SemaphoreType.DMA,
        ],
    )(xp, pos_kw, g16_kw)

    eo = pl.pallas_call(
        _ffn_body,
        grid_spec=pltpu.PrefetchScalarGridSpec(
            num_scalar_prefetch=1,
            grid=(MT,),
            in_specs=[
                pl.BlockSpec((T, D // 2), lambda m, meta: (meta[2, m], 0)),
                pl.BlockSpec((None, D, H), lambda m, meta: (meta[0, m], 0, 0)),
                pl.BlockSpec((None, 1, H), lambda m, meta: (meta[0, m], 0, 0)),
                pl.BlockSpec((None, H, D), lambda m, meta: (meta[0, m], 0, 0)),
                pl.BlockSpec((None, 1, D), lambda m, meta: (meta[0, m], 0, 0)),
                pl.BlockSpec((T, 128), lambda m, meta: (meta[2, m], 0)),
            ],
            out_specs=pl.BlockSpec((T, D), lambda m, meta: (meta[2, m], 0)),
        ),
        out_shape=jax.ShapeDtypeStruct((P, D), jnp.float32),
    )(meta, xs, W1, b1[:, None, :], W2, b2[:, None, :], gs)

    out = pl.kernel(
        _combine_body,
        out_type=jax.ShapeDtypeStruct((B, D), jnp.float32),
        mesh=mesh,
        scratch_types=[
            pltpu.VMEM((CH, D), jnp.float32),
            pltpu.VMEM((CH, D), jnp.float32),
            pltpu.VMEM((CH, D), jnp.float32),
            pltpu.VMEM((CH, D), jnp.float32),
            pltpu.VMEM((TPW,), jnp.int32),
            pltpu.VMEM((TPW,), jnp.int32),
            pltpu.SemaphoreType.DMA,
            pltpu.SemaphoreType.DMA,
        ],
    )(eo, pos_kw)

    return out, probs, tki


def kernel(x, gate_W, gate_b, W1, b1, W2, b2):
    return _moe(x, gate_W, gate_b, W1, b1, W2, b2)


# speedup vs baseline: 3.2718x; 1.0833x over previous
import functools

import jax
import jax.numpy as jnp
from jax import lax
from jax.experimental import pallas as pl
from jax.experimental.pallas import tpu as pltpu
from jax.experimental.pallas import tpu_sc as plsc

E = 8
K = 2
D = 1024
H = 1024
B = 2048

T = 256
MT = 24
P = MT * T
NC = 2
NS = 16
NW = NC * NS
TPW = B // NW
NEG = -1e30


def _gating_body(x_ref, gw_ref, gb_ref, probs_ref, tki_ref, pos_ref,
                 g16_ref, xp_ref, meta_ref):
    x = x_ref[...]
    xb = x.astype(jnp.bfloat16)
    lo = lax.bitcast_convert_type(xb[:, :D // 2], jnp.uint16)
    hi = lax.bitcast_convert_type(xb[:, D // 2:], jnp.uint16)
    packed = lo.astype(jnp.uint32) | (hi.astype(jnp.uint32) << 16)
    xp_ref[...] = lax.bitcast_convert_type(packed, jnp.int32)
    logits = jnp.dot(x, gw_ref[...], preferred_element_type=jnp.float32)
    logits = logits + gb_ref[...][None, :]
    m = jnp.max(logits, axis=1, keepdims=True)
    ex = jnp.exp(logits - m)
    probs_ref[...] = ex / jnp.sum(ex, axis=1, keepdims=True)

    idx = lax.broadcasted_iota(jnp.int32, (B, E), 1)
    i1 = jnp.min(jnp.where(logits == m, idx, E), axis=1, keepdims=True)
    oh1 = idx == i1
    masked = jnp.where(oh1, NEG, logits)
    m2 = jnp.max(masked, axis=1, keepdims=True)
    i2 = jnp.min(jnp.where(masked == m2, idx, E), axis=1, keepdims=True)
    oh2 = idx == i2
    tki_ref[...] = jnp.concatenate([i1, i2], axis=1)

    b = jnp.exp(m2 - m)
    g1 = 1.0 / (1.0 + b)
    g2 = b / (1.0 + b)
    ones16 = jnp.ones((1, 128), jnp.float32)
    g16_ref[...] = jnp.concatenate([g1 * ones16, g2 * ones16], axis=0)

    cnt = oh1.astype(jnp.int32) + oh2.astype(jnp.int32)
    c = cnt
    s = 1
    while s < B:
        c = c + jnp.concatenate(
            [jnp.zeros((s, E), jnp.int32), c[:-s]], axis=0)
        s *= 2
    counts = c[B - 1:B, :]
    tiles_e = (counts + (T - 1)) // T
    tin = tiles_e
    s = 1
    while s < E:
        tin = tin + jnp.concatenate(
            [jnp.zeros((1, s), jnp.int32), tin[:, :-s]], axis=1)
        s *= 2
    toff = tin - tiles_e
    num_tiles = tin[:, E - 1:E]

    rank1 = jnp.sum(c * oh1, axis=1, keepdims=True) - 1
    rank2 = jnp.sum(c * oh2, axis=1, keepdims=True) - 1
    base1 = jnp.sum(jnp.broadcast_to(toff, (B, E)) * oh1, axis=1,
                    keepdims=True) * T
    base2 = jnp.sum(jnp.broadcast_to(toff, (B, E)) * oh2, axis=1,
                    keepdims=True) * T
    pos_ref[...] = jnp.concatenate([base1 + rank1, base2 + rank2], axis=1)

    miota = lax.broadcasted_iota(jnp.int32, (1, 128), 1)
    eot = jnp.zeros((1, 128), jnp.int32)
    for e in range(E):
        eot = eot + (miota >= toff[0, e]).astype(jnp.int32)
    eot = eot - 1
    real = (miota < num_tiles).astype(jnp.int32)
    xidx = jnp.minimum(miota, num_tiles - 1)
    echg = jnp.concatenate(
        [jnp.ones((1, 1), jnp.int32),
         (eot[:, 1:] != eot[:, :-1]).astype(jnp.int32)], axis=1)
    ri = echg
    s = 1
    while s < 128:
        ri = ri + jnp.concatenate(
            [jnp.zeros((1, s), jnp.int32), ri[:, :-s]], axis=1)
        s *= 2
    parity = jnp.bitwise_and(ri - 1, 1)
    nz = counts > 0
    succ = [None] * E
    succ[E - 1] = jnp.full((1, 1), -1, jnp.int32)
    for e in range(E - 2, -1, -1):
        succ[e] = jnp.where(nz[:, e + 1:e + 2], e + 1, succ[e + 1])
    nexte_t = jnp.zeros((1, 128), jnp.int32)
    for e in range(E):
        ne = jnp.where(succ[e][0, 0] < 0, e, succ[e][0, 0])
        nexte_t = nexte_t + (eot == e).astype(jnp.int32) * ne
    meta_ref[...] = jnp.concatenate(
        [eot, real, xidx, echg, parity, nexte_t,
         jnp.zeros((2, 128), jnp.int32)], axis=0)


def _dispatch_body(x_hbm, pos_hbm, g16_hbm, xs_hbm, gs_hbm,
                   xv, idx0v, idx1v, g16v0, g16v1, sem):
    wid = lax.axis_index("s") * NC + lax.axis_index("c")
    base = wid * TPW
    pltpu.sync_copy(pos_hbm.at[0, wid], idx0v)
    pltpu.sync_copy(pos_hbm.at[1, wid], idx1v)
    pltpu.sync_copy(g16_hbm.at[0, wid], g16v0)
    pltpu.sync_copy(g16_hbm.at[1, wid], g16v1)
    pltpu.sync_copy(x_hbm.at[pl.ds(base, TPW)], xv)
    c0 = pltpu.async_copy(xv, xs_hbm.at[idx0v], sem)
    c1 = pltpu.async_copy(xv, xs_hbm.at[idx1v], sem)
    c2 = pltpu.async_copy(g16v0, gs_hbm.at[idx0v], sem)
    c3 = pltpu.async_copy(g16v1, gs_hbm.at[idx1v], sem)
    c0.wait()
    c1.wait()
    c2.wait()
    c3.wait()


def _ffn_body(meta_ref, xs_ref, w1_hbm, b1_ref, w2_hbm, b2_ref, gs_ref,
              out_ref, w1a, w2a, w1b, w2b, sema, semb):
    m = pl.program_id(0)
    ecur = meta_ref[0, m]
    first = meta_ref[3, m] == 1
    par = meta_ref[4, m]
    nxt = meta_ref[5, m]

    @pl.when(m == 0)
    def _():
        pltpu.make_async_copy(w1_hbm.at[ecur], w1a, sema).start()
        pltpu.make_async_copy(w2_hbm.at[ecur], w2a, sema).start()

    @pl.when(first & (par == 0))
    def _():
        pltpu.make_async_copy(w1_hbm.at[ecur], w1a, sema).wait()
        pltpu.make_async_copy(w2_hbm.at[ecur], w2a, sema).wait()

        @pl.when(nxt != ecur)
        def _():
            pltpu.make_async_copy(w1_hbm.at[nxt], w1b, semb).start()
            pltpu.make_async_copy(w2_hbm.at[nxt], w2b, semb).start()

    @pl.when(first & (par == 1))
    def _():
        pltpu.make_async_copy(w1_hbm.at[ecur], w1b, semb).wait()
        pltpu.make_async_copy(w2_hbm.at[ecur], w2b, semb).wait()

        @pl.when(nxt != ecur)
        def _():
            pltpu.make_async_copy(w1_hbm.at[nxt], w1a, sema).start()
            pltpu.make_async_copy(w2_hbm.at[nxt], w2a, sema).start()

    def compute(w1_ref, w2_ref):
        v = xs_ref[...]
        xlo = lax.bitcast_convert_type(
            (v & 0xFFFF).astype(jnp.uint16), jnp.bfloat16)
        xhi = lax.bitcast_convert_type(
            lax.shift_right_logical(v, 16).astype(jnp.uint16), jnp.bfloat16)
        w1lo = w1_ref[: D // 2, :].astype(jnp.bfloat16)
        w1hi = w1_ref[D // 2:, :].astype(jnp.bfloat16)
        h = (jnp.dot(xlo, w1lo, preferred_element_type=jnp.float32)
             + jnp.dot(xhi, w1hi, preferred_element_type=jnp.float32))
        h = jnp.maximum(h + b1_ref[0][None, :], 0.0).astype(jnp.bfloat16)
        w2 = w2_ref[...].astype(jnp.bfloat16)
        eo = jnp.dot(h, w2, preferred_element_type=jnp.float32)
        out_ref[...] = (eo + b2_ref[0][None, :]) * gs_ref[...][:, 0:1]

    real = meta_ref[1, m] == 1

    @pl.when(real & (par == 0))
    def _():
        compute(w1a, w2a)

    @pl.when(real & (par == 1))
    def _():
        compute(w1b, w2b)


CH = 16
NCH = TPW // CH


def _combine_body(eo_hbm, pos_hbm, out_hbm, av0, bv0, av1, bv1,
                  idx0v, idx1v, sem0, sem1):
    wid = lax.axis_index("s") * NC + lax.axis_index("c")
    base = wid * TPW
    pltpu.sync_copy(pos_hbm.at[0, wid], idx0v)
    pltpu.sync_copy(pos_hbm.at[1, wid], idx1v)
    avs = [av0, av1]
    bvs = [bv0, bv1]
    nsl = D // 16

    def fire(h):
        c0 = pltpu.async_copy(
            eo_hbm.at[idx0v.at[pl.ds(CH * h, CH)]], avs[h % 2], sem0)
        c1 = pltpu.async_copy(
            eo_hbm.at[idx1v.at[pl.ds(CH * h, CH)]], bvs[h % 2], sem1)
        return c0, c1

    pend = fire(0)
    for h in range(NCH):
        pend[0].wait()
        pend[1].wait()
        if h + 1 < NCH:
            pend = fire(h + 1)
        av = avs[h % 2]
        bv = bvs[h % 2]

        @plsc.parallel_loop(0, CH * nsl, unroll=8)
        def _add(j):
            r = j // nsl
            sl = pl.ds(16 * (j % nsl), 16)
            av[r, sl] = av[r, sl] + bv[r, sl]

        pltpu.sync_copy(av, out_hbm.at[pl.ds(base + CH * h, CH)])


@jax.jit
def _moe(x, gate_W, gate_b, W1, b1, W2, b2):
    probs, tki, pos, g16, xp, meta = pl.pallas_call(
        _gating_body,
        out_shape=(
            jax.ShapeDtypeStruct((B, E), jnp.float32),
            jax.ShapeDtypeStruct((B, K), jnp.int32),
            jax.ShapeDtypeStruct((B, K), jnp.int32),
            jax.ShapeDtypeStruct((K * B, 128), jnp.float32),
            jax.ShapeDtypeStruct((B, D // 2), jnp.int32),
            jax.ShapeDtypeStruct((8, 128), jnp.int32),
        ),
    )(x, gate_W, gate_b)

    pos_kw = pos.T.reshape(K, NW, TPW)
    g16_kw = g16.reshape(K, NW, TPW, 128)

    mesh = plsc.VectorSubcoreMesh(core_axis_name="c", subcore_axis_name="s")
    xs, gs = pl.kernel(
        _dispatch_body,
        out_type=(
            jax.ShapeDtypeStruct((P, D // 2), jnp.int32),
            jax.ShapeDtypeStruct((P, 128), jnp.float32),
        ),
        mesh=mesh,
        scratch_types=[
            pltpu.VMEM((TPW, D // 2), jnp.int32),
            pltpu.VMEM((TPW,), jnp.int32),
            pltpu.VMEM((TPW,), jnp.int32),
            pltpu.VMEM((TPW, 128), jnp.float32),
            pltpu.VMEM((TPW, 128), jnp.float32),
            pltpu.SemaphoreType.DMA,
        ],
    )(xp, pos_kw, g16_kw)

    eo = pl.pallas_call(
        _ffn_body,
        grid_spec=pltpu.PrefetchScalarGridSpec(
            num_scalar_prefetch=1,
            grid=(MT,),
            in_specs=[
                pl.BlockSpec((T, D // 2), lambda m, meta: (meta[2, m], 0)),
                pl.BlockSpec(memory_space=pl.ANY),
                pl.BlockSpec((None, 1, H), lambda m, meta: (meta[0, m], 0, 0)),
                pl.BlockSpec(memory_space=pl.ANY),
                pl.BlockSpec((None, 1, D), lambda m, meta: (meta[0, m], 0, 0)),
                pl.BlockSpec((T, 128), lambda m, meta: (meta[2, m], 0)),
            ],
            out_specs=pl.BlockSpec((T, D), lambda m, meta: (meta[2, m], 0)),
            scratch_shapes=[
                pltpu.VMEM((D, H), jnp.float32),
                pltpu.VMEM((H, D), jnp.float32),
                pltpu.VMEM((D, H), jnp.float32),
                pltpu.VMEM((H, D), jnp.float32),
                pltpu.SemaphoreType.DMA,
                pltpu.SemaphoreType.DMA,
            ],
        ),
        out_shape=jax.ShapeDtypeStruct((P, D), jnp.float32),
    )(meta, xs, W1, b1[:, None, :], W2, b2[:, None, :], gs)

    out = pl.kernel(
        _combine_body,
        out_type=jax.ShapeDtypeStruct((B, D), jnp.float32),
        mesh=mesh,
        scratch_types=[
            pltpu.VMEM((CH, D), jnp.float32),
            pltpu.VMEM((CH, D), jnp.float32),
            pltpu.VMEM((CH, D), jnp.float32),
            pltpu.VMEM((CH, D), jnp.float32),
            pltpu.VMEM((TPW,), jnp.int32),
            pltpu.VMEM((TPW,), jnp.int32),
            pltpu.SemaphoreType.DMA,
            pltpu.SemaphoreType.DMA,
        ],
    )(eo, pos_kw)

    return out, probs, tki


def kernel(x, gate_W, gate_b, W1, b1, W2, b2):
    return _moe(x, gate_W, gate_b, W1, b1, W2, b2)
